# baseline (speedup 1.0000x reference)
import functools

import jax
import jax.numpy as jnp
from jax import lax
from jax.experimental import pallas as pl
from jax.experimental.pallas import tpu as pltpu
from jax.experimental.pallas import tpu_sc as plsc

L = 16
NC, NS = 2, 16
NW = NC * NS
R = 1_000_000
D = 64
B = 16384
CW = 512
CPW = 61
PPW = CPW * CW
TAIL = R - 1953 * CW
MAXP = R - 31 * PPW
NCHUNK = B // L

_mesh = plsc.VectorSubcoreMesh(core_axis_name="c", subcore_axis_name="s")


@functools.partial(
    pl.kernel,
    out_type=jax.ShapeDtypeStruct((D, R), jnp.float32),
    mesh=_mesh,
    compiler_params=pltpu.CompilerParams(
        needs_layout_passes=False, use_tc_tiling_on_sc=True),
    scratch_types=[
        pltpu.VMEM((B,), jnp.int32),
        pltpu.VMEM((MAXP,), jnp.int32),
        pltpu.VMEM((L,), jnp.int32),
        pltpu.VMEM((CW + L,), jnp.int32),
        pltpu.VMEM((CW + L,), jnp.int32),
        pltpu.VMEM((CW + L,), jnp.int32),
        pltpu.VMEM((CW + L,), jnp.int32),
        pltpu.VMEM((D, CW), jnp.float32),
        pltpu.VMEM((D, CW), jnp.float32),
        pltpu.VMEM((L, 128), jnp.float32),
        pltpu.VMEM((L, 128), jnp.float32),
        pltpu.VMEM((D, TAIL), jnp.float32),
        pltpu.SemaphoreType.DMA,
        pltpu.SemaphoreType.DMA,
        pltpu.SemaphoreType.DMA,
        pltpu.SemaphoreType.DMA,
    ],
)
def _sc_index_put_fused(xt_hbm, idx_hbm, vals_pad_hbm, out_ref, idx_v, marker,
                        nh_v, hcol0, hpos0, hcol1, hpos1, blk0, blk1, gbuf0,
                        gbuf1, tblk, lsem, ssem, gsem0, gsem1):
    wid = lax.axis_index("s") * NC + lax.axis_index("c")
    base = wid * PPW
    npos = jnp.where(wid == NW - 1, MAXP, PPW)
    lane = lax.iota(jnp.int32, L)

    def start_load(c, blk):
        pltpu.async_copy(xt_hbm.at[:, pl.ds(base + c * CW, CW)], blk, lsem)

    def wait_load(blk):
        pltpu.make_async_copy(xt_hbm.at[:, pl.ds(0, CW)], blk, lsem).wait()

    start_load(0, blk0)

    pltpu.sync_copy(idx_hbm, idx_v)

    def clear(i, carry):
        marker[pl.ds(i * L, L)] = jnp.full((L,), -1, jnp.int32)
        return carry

    lax.fori_loop(0, (MAXP + L - 1) // L, clear, 0)

    def p1(c, carry):
        i16 = idx_v[pl.ds(c * L, L)]
        local = i16 - base
        member = (i16 >= base) & (local < npos)
        plsc.store_scatter(marker, [local], c * L + lane, mask=member)
        return carry

    lax.fori_loop(0, NCHUNK, p1, 0)


    def padded_rows(hpos, gbase, nhits):
        m16 = hpos[pl.ds(gbase, L)]
        valid = lane < (nhits - gbase)
        return jnp.where(valid, m16, jnp.broadcast_to(m16[0], (L,)))

    def scan_hits(moff, width, hcol, hpos):
        def scan(k, nh):
            m16 = marker[pl.ds(moff + k * L, L)]
            hit = m16 >= 0
            plsc.store_compressed(hcol.at[pl.ds(nh, L)], k * L + lane,
                                  mask=hit)
            plsc.store_compressed(hpos.at[pl.ds(nh, L)], m16, mask=hit)
            return nh + jnp.sum(hit.astype(jnp.int32))

        return lax.fori_loop(0, width // L, scan, jnp.int32(0))

    def fire_gather0(nhits, hpos, gbuf, gsem):
        @pl.when(nhits > 0)
        def _():
            pltpu.async_copy(
                vals_pad_hbm.at[padded_rows(hpos, 0, nhits)], gbuf, gsem)

    def prep(c, hcol, hpos, gbuf, gsem):
        nhits = scan_hits(c * CW, CW, hcol, hpos)
        fire_gather0(nhits, hpos, gbuf, gsem)
        nh_v[pl.ds(0, L)] = jnp.broadcast_to(nhits, (L,)).astype(jnp.int32)

    def apply_hits(blk, nhits, hcol, hpos, gbuf, gsem):
        def group(g, carry):
            gbase = g * L

            @pl.when(g == 0)
            def _():
                pltpu.make_async_copy(
                    vals_pad_hbm.at[lane], gbuf, gsem).wait()

            @pl.when(g >= 1)
            def _():
                pltpu.async_copy(
                    vals_pad_hbm.at[padded_rows(hpos, gbase, nhits)], gbuf,
                    gsem).wait()

            cols16 = hcol[pl.ds(gbase, L)]
            valid = lane < (nhits - gbase)
            cols16 = jnp.where(valid, cols16, jnp.zeros((L,), jnp.int32))
            for k in range(D):
                kvec = jnp.full((L,), k, jnp.int32)
                vk = plsc.load_gather(gbuf, [lane, kvec])
                plsc.store_scatter(blk, [kvec, cols16], vk, mask=valid)
            return carry

        lax.fori_loop(0, (nhits + L - 1) // L, group, 0)

    def patch_sync(blk, moff, width):
        nhits = scan_hits(moff, width, hcol0, hpos0)
        fire_gather0(nhits, hpos0, gbuf0, gsem0)
        apply_hits(blk, nhits, hcol0, hpos0, gbuf0, gsem0)


    prep(0, hcol0, hpos0, gbuf0, gsem0)

    def step(c, carry):
        off = base + c * CW
        nhits_c = nh_v[pl.ds(0, L)][0]

        @pl.when(c >= 1)
        def _():
            pltpu.make_async_copy(blk0, out_ref.at[:, pl.ds(0, CW)],
                                  ssem).wait()

        @pl.when(c % 2 == 0)
        def _():
            @pl.when(c + 1 < CPW)
            def _():
                prep(c + 1, hcol1, hpos1, gbuf1, gsem1)

            wait_load(blk0)

            @pl.when(c + 1 < CPW)
            def _():
                start_load(c + 1, blk1)

            apply_hits(blk0, nhits_c, hcol0, hpos0, gbuf0, gsem0)
            pltpu.async_copy(blk0, out_ref.at[:, pl.ds(off, CW)], ssem)

        @pl.when(c % 2 == 1)
        def _():
            @pl.when(c + 1 < CPW)
            def _():
                prep(c + 1, hcol0, hpos0, gbuf0, gsem0)

            wait_load(blk1)

            @pl.when(c + 1 < CPW)
            def _():
                start_load(c + 1, blk0)

            apply_hits(blk1, nhits_c, hcol1, hpos1, gbuf1, gsem1)
            pltpu.async_copy(blk1, out_ref.at[:, pl.ds(off, CW)], ssem)

        return carry

    lax.fori_loop(0, CPW, step, 0)
    pltpu.make_async_copy(blk0, out_ref.at[:, pl.ds(0, CW)], ssem).wait()

    @pl.when(wid == NW - 1)
    def _():
        off = 1952 * CW
        pltpu.sync_copy(xt_hbm.at[:, pl.ds(off, CW)], blk0)
        patch_sync(blk0, off - base, CW)
        pltpu.sync_copy(blk0, out_ref.at[:, pl.ds(off, CW)])

        toff = 1953 * CW
        pltpu.sync_copy(xt_hbm.at[:, pl.ds(toff, TAIL)], tblk)
        patch_sync(tblk, toff - base, TAIL)
        pltpu.sync_copy(tblk, out_ref.at[:, pl.ds(toff, TAIL)])


def kernel(x, idx, vals):
    vals_pad = jnp.pad(vals, ((0, 0), (0, 128 - D)))
    out_t = _sc_index_put_fused(x.T, idx.astype(jnp.int32), vals_pad)
    return out_t.T


# speedup vs baseline: 1.0114x; 1.0114x over previous
import functools

import jax
import jax.numpy as jnp
from jax import lax
from jax.experimental import pallas as pl
from jax.experimental.pallas import tpu as pltpu
from jax.experimental.pallas import tpu_sc as plsc

L = 16
NC, NS = 2, 16
NW = NC * NS
R = 1_000_000
D = 64
B = 16384
CW = 512
CPW = 61
PPW = CPW * CW
TAIL = R - 1953 * CW
MAXP = R - 31 * PPW
NCHUNK = B // L

_mesh = plsc.VectorSubcoreMesh(core_axis_name="c", subcore_axis_name="s")


@functools.partial(
    pl.kernel,
    out_type=jax.ShapeDtypeStruct((D, R), jnp.float32),
    mesh=_mesh,
    compiler_params=pltpu.CompilerParams(
        needs_layout_passes=False, use_tc_tiling_on_sc=True),
    scratch_types=[
        pltpu.VMEM((B,), jnp.int32),
        pltpu.VMEM((MAXP,), jnp.int32),
        pltpu.VMEM((L,), jnp.int32),
        pltpu.VMEM((CW + L,), jnp.int32),
        pltpu.VMEM((CW + L,), jnp.int32),
        pltpu.VMEM((CW + L,), jnp.int32),
        pltpu.VMEM((CW + L,), jnp.int32),
        pltpu.VMEM((D, CW), jnp.float32),
        pltpu.VMEM((D, CW), jnp.float32),
        pltpu.VMEM((L, 128), jnp.float32),
        pltpu.VMEM((L, 128), jnp.float32),
        pltpu.VMEM((D, TAIL), jnp.float32),
        pltpu.SemaphoreType.DMA,
        pltpu.SemaphoreType.DMA,
        pltpu.SemaphoreType.DMA,
        pltpu.SemaphoreType.DMA,
    ],
)
def _sc_index_put_fused(xt_hbm, idx_hbm, vals_pad_hbm, out_ref, idx_v, marker,
                        nh_v, hcol0, hpos0, hcol1, hpos1, blk0, blk1, gbuf0,
                        gbuf1, tblk, lsem, ssem, gsem0, gsem1):
    wid = lax.axis_index("s") * NC + lax.axis_index("c")
    base = wid * PPW
    npos = jnp.where(wid == NW - 1, MAXP, PPW)
    lane = lax.iota(jnp.int32, L)

    def start_load(c, blk):
        pltpu.async_copy(xt_hbm.at[:, pl.ds(base + c * CW, CW)], blk, lsem)

    def wait_load(blk):
        pltpu.make_async_copy(xt_hbm.at[:, pl.ds(0, CW)], blk, lsem).wait()

    start_load(0, blk0)

    pltpu.sync_copy(idx_hbm, idx_v)

    def clear(i, carry):
        marker[pl.ds(i * L, L)] = jnp.full((L,), -1, jnp.int32)
        return carry

    lax.fori_loop(0, (MAXP + L - 1) // L, clear, 0)

    def p1(c, carry):
        i16 = idx_v[pl.ds(c * L, L)]
        local = i16 - base
        member = (i16 >= base) & (local < npos)
        plsc.store_scatter(marker, [local], c * L + lane, mask=member)
        return carry

    lax.fori_loop(0, NCHUNK, p1, 0)


    def padded_rows(hpos, gbase, nhits):
        m16 = hpos[pl.ds(gbase, L)]
        valid = lane < (nhits - gbase)
        return jnp.where(valid, m16, jnp.broadcast_to(m16[0], (L,)))

    def scan_hits(moff, width, hcol, hpos):
        def scan(k, nh):
            m16 = marker[pl.ds(moff + k * L, L)]
            hit = m16 >= 0
            plsc.store_compressed(hcol.at[pl.ds(nh, L)], k * L + lane,
                                  mask=hit)
            plsc.store_compressed(hpos.at[pl.ds(nh, L)], m16, mask=hit)
            return nh + jnp.sum(hit.astype(jnp.int32))

        return lax.fori_loop(0, width // L, scan, jnp.int32(0))

    def fire_gather0(nhits, hpos, gbuf, gsem):
        @pl.when(nhits > 0)
        def _():
            pltpu.async_copy(
                vals_pad_hbm.at[padded_rows(hpos, 0, nhits)], gbuf, gsem)

    def prep(c, hcol, hpos, gbuf, gsem):
        nhits = scan_hits(c * CW, CW, hcol, hpos)
        fire_gather0(nhits, hpos, gbuf, gsem)
        nh_v[pl.ds(0, L)] = jnp.broadcast_to(nhits, (L,)).astype(jnp.int32)

    def apply_hits(blk, nhits, hcol, hpos, gbuf, gsem):
        def group(g, carry):
            gbase = g * L

            @pl.when(g == 0)
            def _():
                pltpu.make_async_copy(
                    vals_pad_hbm.at[lane], gbuf, gsem).wait()

            @pl.when(g >= 1)
            def _():
                pltpu.async_copy(
                    vals_pad_hbm.at[padded_rows(hpos, gbase, nhits)], gbuf,
                    gsem).wait()

            def one(l, carry2):
                col = hcol[pl.ds(gbase + l, L)][0]
                cvec = jnp.broadcast_to(col, (L,)).astype(jnp.int32)
                lvec = jnp.broadcast_to(l, (L,)).astype(jnp.int32)
                for k in range(D // L):
                    v = plsc.load_gather(gbuf, [lvec, k * L + lane])
                    plsc.store_scatter(blk, [k * L + lane, cvec], v)
                return carry2

            lax.fori_loop(0, jnp.minimum(nhits - gbase, L), one, 0)
            return carry

        lax.fori_loop(0, (nhits + L - 1) // L, group, 0)

    def patch_sync(blk, moff, width):
        nhits = scan_hits(moff, width, hcol0, hpos0)
        fire_gather0(nhits, hpos0, gbuf0, gsem0)
        apply_hits(blk, nhits, hcol0, hpos0, gbuf0, gsem0)


    prep(0, hcol0, hpos0, gbuf0, gsem0)

    def step(c, carry):
        off = base + c * CW
        nhits_c = nh_v[pl.ds(0, L)][0]

        @pl.when(c >= 1)
        def _():
            pltpu.make_async_copy(blk0, out_ref.at[:, pl.ds(0, CW)],
                                  ssem).wait()

        @pl.when(c % 2 == 0)
        def _():
            @pl.when(c + 1 < CPW)
            def _():
                prep(c + 1, hcol1, hpos1, gbuf1, gsem1)

            wait_load(blk0)

            @pl.when(c + 1 < CPW)
            def _():
                start_load(c + 1, blk1)

            apply_hits(blk0, nhits_c, hcol0, hpos0, gbuf0, gsem0)
            pltpu.async_copy(blk0, out_ref.at[:, pl.ds(off, CW)], ssem)

        @pl.when(c % 2 == 1)
        def _():
            @pl.when(c + 1 < CPW)
            def _():
                prep(c + 1, hcol0, hpos0, gbuf0, gsem0)

            wait_load(blk1)

            @pl.when(c + 1 < CPW)
            def _():
                start_load(c + 1, blk0)

            apply_hits(blk1, nhits_c, hcol1, hpos1, gbuf1, gsem1)
            pltpu.async_copy(blk1, out_ref.at[:, pl.ds(off, CW)], ssem)

        return carry

    lax.fori_loop(0, CPW, step, 0)
    pltpu.make_async_copy(blk0, out_ref.at[:, pl.ds(0, CW)], ssem).wait()

    @pl.when(wid == NW - 1)
    def _():
        off = 1952 * CW
        pltpu.sync_copy(xt_hbm.at[:, pl.ds(off, CW)], blk0)
        patch_sync(blk0, off - base, CW)
        pltpu.sync_copy(blk0, out_ref.at[:, pl.ds(off, CW)])

        toff = 1953 * CW
        pltpu.sync_copy(xt_hbm.at[:, pl.ds(toff, TAIL)], tblk)
        patch_sync(tblk, toff - base, TAIL)
        pltpu.sync_copy(tblk, out_ref.at[:, pl.ds(toff, TAIL)])


def kernel(x, idx, vals):
    vals_pad = jnp.pad(vals, ((0, 0), (0, 128 - D)))
    out_t = _sc_index_put_fused(x.T, idx.astype(jnp.int32), vals_pad)
    return out_t.T
